# Initial kernel scaffold; baseline (speedup 1.0000x reference)
import jax
import jax.numpy as jnp
from jax.experimental import pallas as pl


def kernel(gc1_w, gc1_b, gc2_w, gc2_b, gc3_mean_w, gc3_mean_b, gc3_logstd_w, gc3_logstd_b, gc4_mean_w, gc4_mean_b, gc4_logstd_w, gc4_logstd_b, union_source_mean_w, union_source_mean_b, union_source_logstd_w, union_source_logstd_b, union_target_mean_w, union_target_mean_b, union_target_logstd_w, union_target_logstd_b, source_ufea, target_ufea, source_UV_adj, source_VU_adj, target_UV_adj, target_VU_adj):
    raise NotImplementedError("write your pallas kernel here")


# speedup vs baseline: 1.1312x; 1.1312x over previous
import functools

import jax
import jax.numpy as jnp
from jax.experimental import pallas as pl
from jax.experimental.pallas import tpu as pltpu

_ALPHA = 0.1
_RATE = 0.7

_TM1 = 256
_TM2 = 256
_VMEM = 48 * 1024 * 1024


def _leaky(v):
    return jnp.where(v > 0.0, v, _ALPHA * v)


def _layer1_body(adj_ref, x_ref, w1_ref, b1_ref, w3_ref, o_ref):
    sup1 = jnp.dot(x_ref[...], w1_ref[...], preferred_element_type=jnp.float32)
    acc = jnp.dot(adj_ref[...], sup1, preferred_element_type=jnp.float32)
    h = _leaky(acc + b1_ref[...])
    o_ref[...] = jnp.dot(h, w3_ref[...], preferred_element_type=jnp.float32)


def _layer1(adj, x, w1, b1, w3cat):
    n_rows, n_k = adj.shape
    n_hid = w1.shape[1]
    n_out = w3cat.shape[1]
    tm = min(_TM1, n_rows)
    return pl.pallas_call(
        _layer1_body,
        grid=(n_rows // tm,),
        in_specs=[
            pl.BlockSpec((tm, n_k), lambda i: (i, 0)),
            pl.BlockSpec((n_k, x.shape[1]), lambda i: (0, 0)),
            pl.BlockSpec((x.shape[1], n_hid), lambda i: (0, 0)),
            pl.BlockSpec((1, n_hid), lambda i: (0, 0)),
            pl.BlockSpec((n_hid, n_out), lambda i: (0, 0)),
        ],
        out_specs=pl.BlockSpec((tm, n_out), lambda i: (i, 0)),
        out_shape=jax.ShapeDtypeStruct((n_rows, n_out), jnp.float32),
        compiler_params=pltpu.CompilerParams(
            dimension_semantics=("parallel",),
            vmem_limit_bytes=_VMEM,
        ),
    )(adj, x, w1, b1.reshape(1, -1), w3cat)


def _layer2_union_body(adj_s_ref, adj_t_ref, sup_s_ref, sup_t_ref,
                       b3_ref, b4_ref, sf_ref, tf_ref,
                       wsc_ref, wsf_ref, wtc_ref, wtf_ref, bu_ref,
                       om_ref, ol_ref, *, fdim):
    s_cat = _leaky(
        jnp.dot(adj_s_ref[...], sup_s_ref[...], preferred_element_type=jnp.float32)
        + b3_ref[...])
    t_cat = _leaky(
        jnp.dot(adj_t_ref[...], sup_t_ref[...], preferred_element_type=jnp.float32)
        + b4_ref[...])
    out = jnp.dot(s_cat, wsc_ref[...], preferred_element_type=jnp.float32)
    out = out + jnp.dot(sf_ref[...], wsf_ref[...], preferred_element_type=jnp.float32)
    out = out + jnp.dot(t_cat, wtc_ref[...], preferred_element_type=jnp.float32)
    out = out + jnp.dot(tf_ref[...], wtf_ref[...], preferred_element_type=jnp.float32)
    out = out + bu_ref[...]
    om_ref[...] = out[:, :fdim]
    ol_ref[...] = out[:, fdim:]


def kernel(gc1_w, gc1_b, gc2_w, gc2_b,
           gc3_mean_w, gc3_mean_b, gc3_logstd_w, gc3_logstd_b,
           gc4_mean_w, gc4_mean_b, gc4_logstd_w, gc4_logstd_b,
           union_source_mean_w, union_source_mean_b,
           union_source_logstd_w, union_source_logstd_b,
           union_target_mean_w, union_target_mean_b,
           union_target_logstd_w, union_target_logstd_b,
           source_ufea, target_ufea,
           source_UV_adj, source_VU_adj, target_UV_adj, target_VU_adj):
    fdim = source_ufea.shape[1]
    n_user = source_ufea.shape[0]
    two_f = 2 * fdim

    w3 = jnp.concatenate([gc3_mean_w, gc3_logstd_w], axis=1)
    b3 = jnp.concatenate([gc3_mean_b, gc3_logstd_b])
    w4 = jnp.concatenate([gc4_mean_w, gc4_logstd_w], axis=1)
    b4 = jnp.concatenate([gc4_mean_b, gc4_logstd_b])

    sup_s = _layer1(source_VU_adj, source_ufea, gc1_w, gc1_b, w3)
    sup_t = _layer1(target_VU_adj, target_ufea, gc2_w, gc2_b, w4)

    def _split(w):
        return w[:, :fdim].T, w[:, fdim:].T

    wh_sm, wf_sm = _split(union_source_mean_w)
    wh_sl, wf_sl = _split(union_source_logstd_w)
    wh_tm, wf_tm = _split(union_target_mean_w)
    wh_tl, wf_tl = _split(union_target_logstd_w)

    zeros = jnp.zeros((fdim, fdim), jnp.float32)
    rate = jnp.float32(_RATE)
    w_sc = jnp.block([[wh_sm, zeros], [zeros, wh_sl]]) * rate
    w_tc = jnp.block([[wh_tm, zeros], [zeros, wh_tl]]) * (1.0 - rate)
    w_sf = jnp.concatenate([wf_sm, wf_sl], axis=1) * rate
    w_tf = jnp.concatenate([wf_tm, wf_tl], axis=1) * (1.0 - rate)
    b_u = (rate * jnp.concatenate([union_source_mean_b, union_source_logstd_b])
           + (1.0 - rate) * jnp.concatenate([union_target_mean_b,
                                             union_target_logstd_b]))

    n_item_s = source_UV_adj.shape[1]
    n_item_t = target_UV_adj.shape[1]

    tm2 = min(_TM2, n_user)
    mean, logstd = pl.pallas_call(
        functools.partial(_layer2_union_body, fdim=fdim),
        grid=(n_user // tm2,),
        in_specs=[
            pl.BlockSpec((tm2, n_item_s), lambda i: (i, 0)),
            pl.BlockSpec((tm2, n_item_t), lambda i: (i, 0)),
            pl.BlockSpec((n_item_s, two_f), lambda i: (0, 0)),
            pl.BlockSpec((n_item_t, two_f), lambda i: (0, 0)),
            pl.BlockSpec((1, two_f), lambda i: (0, 0)),
            pl.BlockSpec((1, two_f), lambda i: (0, 0)),
            pl.BlockSpec((tm2, fdim), lambda i: (i, 0)),
            pl.BlockSpec((tm2, fdim), lambda i: (i, 0)),
            pl.BlockSpec((two_f, two_f), lambda i: (0, 0)),
            pl.BlockSpec((fdim, two_f), lambda i: (0, 0)),
            pl.BlockSpec((two_f, two_f), lambda i: (0, 0)),
            pl.BlockSpec((fdim, two_f), lambda i: (0, 0)),
            pl.BlockSpec((1, two_f), lambda i: (0, 0)),
        ],
        out_specs=[
            pl.BlockSpec((tm2, fdim), lambda i: (i, 0)),
            pl.BlockSpec((tm2, fdim), lambda i: (i, 0)),
        ],
        out_shape=[
            jax.ShapeDtypeStruct((n_user, fdim), jnp.float32),
            jax.ShapeDtypeStruct((n_user, fdim), jnp.float32),
        ],
        compiler_params=pltpu.CompilerParams(
            dimension_semantics=("parallel",),
            vmem_limit_bytes=_VMEM,
        ),
    )(source_UV_adj, target_UV_adj, sup_s, sup_t,
      b3.reshape(1, -1), b4.reshape(1, -1),
      source_ufea, target_ufea,
      w_sc, w_sf, w_tc, w_tf, b_u.reshape(1, -1))

    return mean, logstd


# speedup vs baseline: 1.1869x; 1.0493x over previous
import functools

import jax
import jax.numpy as jnp
from jax.experimental import pallas as pl
from jax.experimental.pallas import tpu as pltpu

_ALPHA = 0.1
_RATE = 0.7

_TM1 = 512
_TM2 = 512
_VMEM = 48 * 1024 * 1024


def _leaky(v):
    return jnp.where(v > 0.0, v, _ALPHA * v)


def _layer1_body(adj_ref, x_ref, w1_ref, b1_ref, w3_ref, o_ref):
    sup1 = jnp.dot(x_ref[...], w1_ref[...], preferred_element_type=jnp.float32)
    acc = jnp.dot(adj_ref[...], sup1, preferred_element_type=jnp.float32)
    h = _leaky(acc + b1_ref[...])
    o_ref[...] = jnp.dot(h, w3_ref[...], preferred_element_type=jnp.float32)


def _layer1(adj, x, w1, b1, w3cat):
    n_rows, n_k = adj.shape
    n_hid = w1.shape[1]
    n_out = w3cat.shape[1]
    tm = min(_TM1, n_rows)
    return pl.pallas_call(
        _layer1_body,
        grid=(n_rows // tm,),
        in_specs=[
            pl.BlockSpec((tm, n_k), lambda i: (i, 0)),
            pl.BlockSpec((n_k, x.shape[1]), lambda i: (0, 0)),
            pl.BlockSpec((x.shape[1], n_hid), lambda i: (0, 0)),
            pl.BlockSpec((1, n_hid), lambda i: (0, 0)),
            pl.BlockSpec((n_hid, n_out), lambda i: (0, 0)),
        ],
        out_specs=pl.BlockSpec((tm, n_out), lambda i: (i, 0)),
        out_shape=jax.ShapeDtypeStruct((n_rows, n_out), jnp.float32),
        compiler_params=pltpu.CompilerParams(
            dimension_semantics=("parallel",),
            vmem_limit_bytes=_VMEM,
        ),
    )(adj, x, w1, b1.reshape(1, -1), w3cat)


def _layer2_union_body(adj_s_ref, adj_t_ref, sup_s_ref, sup_t_ref,
                       b3_ref, b4_ref, sf_ref, tf_ref,
                       wsc_ref, wsf_ref, wtc_ref, wtf_ref, bu_ref,
                       om_ref, ol_ref, *, fdim):
    s_cat = _leaky(
        jnp.dot(adj_s_ref[...], sup_s_ref[...], preferred_element_type=jnp.float32)
        + b3_ref[...])
    t_cat = _leaky(
        jnp.dot(adj_t_ref[...], sup_t_ref[...], preferred_element_type=jnp.float32)
        + b4_ref[...])
    out = jnp.dot(s_cat, wsc_ref[...], preferred_element_type=jnp.float32)
    out = out + jnp.dot(sf_ref[...], wsf_ref[...], preferred_element_type=jnp.float32)
    out = out + jnp.dot(t_cat, wtc_ref[...], preferred_element_type=jnp.float32)
    out = out + jnp.dot(tf_ref[...], wtf_ref[...], preferred_element_type=jnp.float32)
    out = out + bu_ref[...]
    om_ref[...] = out[:, :fdim]
    ol_ref[...] = out[:, fdim:]


def kernel(gc1_w, gc1_b, gc2_w, gc2_b,
           gc3_mean_w, gc3_mean_b, gc3_logstd_w, gc3_logstd_b,
           gc4_mean_w, gc4_mean_b, gc4_logstd_w, gc4_logstd_b,
           union_source_mean_w, union_source_mean_b,
           union_source_logstd_w, union_source_logstd_b,
           union_target_mean_w, union_target_mean_b,
           union_target_logstd_w, union_target_logstd_b,
           source_ufea, target_ufea,
           source_UV_adj, source_VU_adj, target_UV_adj, target_VU_adj):
    fdim = source_ufea.shape[1]
    n_user = source_ufea.shape[0]
    two_f = 2 * fdim

    w3 = jnp.concatenate([gc3_mean_w, gc3_logstd_w], axis=1)
    b3 = jnp.concatenate([gc3_mean_b, gc3_logstd_b])
    w4 = jnp.concatenate([gc4_mean_w, gc4_logstd_w], axis=1)
    b4 = jnp.concatenate([gc4_mean_b, gc4_logstd_b])

    sup_s = _layer1(source_VU_adj, source_ufea, gc1_w, gc1_b, w3)
    sup_t = _layer1(target_VU_adj, target_ufea, gc2_w, gc2_b, w4)

    def _split(w):
        return w[:, :fdim].T, w[:, fdim:].T

    wh_sm, wf_sm = _split(union_source_mean_w)
    wh_sl, wf_sl = _split(union_source_logstd_w)
    wh_tm, wf_tm = _split(union_target_mean_w)
    wh_tl, wf_tl = _split(union_target_logstd_w)

    zeros = jnp.zeros((fdim, fdim), jnp.float32)
    rate = jnp.float32(_RATE)
    w_sc = jnp.block([[wh_sm, zeros], [zeros, wh_sl]]) * rate
    w_tc = jnp.block([[wh_tm, zeros], [zeros, wh_tl]]) * (1.0 - rate)
    w_sf = jnp.concatenate([wf_sm, wf_sl], axis=1) * rate
    w_tf = jnp.concatenate([wf_tm, wf_tl], axis=1) * (1.0 - rate)
    b_u = (rate * jnp.concatenate([union_source_mean_b, union_source_logstd_b])
           + (1.0 - rate) * jnp.concatenate([union_target_mean_b,
                                             union_target_logstd_b]))

    n_item_s = source_UV_adj.shape[1]
    n_item_t = target_UV_adj.shape[1]

    tm2 = min(_TM2, n_user)
    mean, logstd = pl.pallas_call(
        functools.partial(_layer2_union_body, fdim=fdim),
        grid=(n_user // tm2,),
        in_specs=[
            pl.BlockSpec((tm2, n_item_s), lambda i: (i, 0)),
            pl.BlockSpec((tm2, n_item_t), lambda i: (i, 0)),
            pl.BlockSpec((n_item_s, two_f), lambda i: (0, 0)),
            pl.BlockSpec((n_item_t, two_f), lambda i: (0, 0)),
            pl.BlockSpec((1, two_f), lambda i: (0, 0)),
            pl.BlockSpec((1, two_f), lambda i: (0, 0)),
            pl.BlockSpec((tm2, fdim), lambda i: (i, 0)),
            pl.BlockSpec((tm2, fdim), lambda i: (i, 0)),
            pl.BlockSpec((two_f, two_f), lambda i: (0, 0)),
            pl.BlockSpec((fdim, two_f), lambda i: (0, 0)),
            pl.BlockSpec((two_f, two_f), lambda i: (0, 0)),
            pl.BlockSpec((fdim, two_f), lambda i: (0, 0)),
            pl.BlockSpec((1, two_f), lambda i: (0, 0)),
        ],
        out_specs=[
            pl.BlockSpec((tm2, fdim), lambda i: (i, 0)),
            pl.BlockSpec((tm2, fdim), lambda i: (i, 0)),
        ],
        out_shape=[
            jax.ShapeDtypeStruct((n_user, fdim), jnp.float32),
            jax.ShapeDtypeStruct((n_user, fdim), jnp.float32),
        ],
        compiler_params=pltpu.CompilerParams(
            dimension_semantics=("parallel",),
            vmem_limit_bytes=_VMEM,
        ),
    )(source_UV_adj, target_UV_adj, sup_s, sup_t,
      b3.reshape(1, -1), b4.reshape(1, -1),
      source_ufea, target_ufea,
      w_sc, w_sf, w_tc, w_tf, b_u.reshape(1, -1))

    return mean, logstd
